# Initial kernel scaffold; baseline (speedup 1.0000x reference)
import jax
import jax.numpy as jnp
from jax.experimental import pallas as pl


def kernel(x, edge_index, edge_attr, batch, W1a, b1a, W1b, b1b, W2a, b2a, W2b, b2b, root1, bias1, root2, bias2, fcW1, fcb1, fcW2, fcb2, fcW3, fcb3, fcW4, fcb4):
    raise NotImplementedError("write your pallas kernel here")


# speedup vs baseline: 2.4276x; 2.4276x over previous
import functools

import jax
import jax.numpy as jnp
import numpy as np
from jax import lax
from jax.experimental import pallas as pl
from jax.experimental.pallas import tpu as pltpu
from jax.experimental.pallas import tpu_sc as plsc

N_NODES = 20000
N_EDGES = 80000
N_GRAPHS = 512
D_NODE = 32
D_EDGE = 16
HID = 16
N_CLASSES = 10

NC = 2
NS = 16
NW = NC * NS
CH = 128

E_PAD = 81920
E_HALF = E_PAD // 2
N_PAD = 20480

EB = 2048
NB = 2048

_f32 = jnp.float32


def _mesh():
    return plsc.VectorSubcoreMesh(core_axis_name="c", subcore_axis_name="s")


_SC_PARAMS = pltpu.CompilerParams(use_tc_tiling_on_sc=False)


def _make_sc_gather(n_rows, d, n_idx):
    b_per_w = n_idx // NW
    n_ch = b_per_w // CH

    @functools.partial(
        pl.kernel,
        out_type=jax.ShapeDtypeStruct((n_idx, d), _f32),
        mesh=_mesh(),
        compiler_params=_SC_PARAMS,
        scratch_types=[
            pltpu.VMEM((n_ch, CH), jnp.int32),
            pltpu.VMEM((b_per_w, d), _f32),
            pltpu.SemaphoreType.DMA,
        ],
    )
    def gather(table_hbm, idx_hbm, out_hbm, idx_v, rows_v, sem):
        wid = lax.axis_index("c") * NS + lax.axis_index("s")
        pltpu.sync_copy(idx_hbm.at[wid], idx_v)
        cps = [
            pltpu.async_copy(table_hbm.at[idx_v.at[j]],
                             rows_v.at[pl.ds(j * CH, CH)], sem)
            for j in range(n_ch)
        ]
        for cp in cps:
            cp.wait()
        pltpu.sync_copy(rows_v, out_hbm.at[pl.ds(wid * b_per_w, b_per_w)])

    return gather


def _make_sc_scatter(n_rows, w, n_seg):
    b_per_w = n_rows // NW
    n_ch = b_per_w // CH
    s_per_t = n_seg // NS

    @functools.partial(
        pl.kernel,
        out_type=jax.ShapeDtypeStruct((NC, n_seg, w), _f32),
        mesh=_mesh(),
        compiler_params=_SC_PARAMS,
        scratch_types=[
            pltpu.VMEM((n_ch, CH), jnp.int32),
            pltpu.VMEM((b_per_w, w), _f32),
            pltpu.VMEM_SHARED((n_seg, w), _f32),
            pltpu.SemaphoreType.DMA,
        ],
    )
    def scatter(data_hbm, idx_hbm, zeros_hbm, out_hbm, idx_v, rows_v, acc_sh, sem):
        c = lax.axis_index("c")
        s = lax.axis_index("s")
        wid = c * NS + s
        pltpu.sync_copy(zeros_hbm, rows_v.at[pl.ds(0, s_per_t)])
        pltpu.sync_copy(rows_v.at[pl.ds(0, s_per_t)],
                        acc_sh.at[pl.ds(s * s_per_t, s_per_t)])
        plsc.subcore_barrier()
        pltpu.sync_copy(idx_hbm.at[wid], idx_v)
        pltpu.sync_copy(data_hbm.at[pl.ds(wid * b_per_w, b_per_w)], rows_v)
        for j in range(n_ch):
            pltpu.sync_copy(rows_v.at[pl.ds(j * CH, CH)],
                            acc_sh.at[idx_v.at[j]], add=True)
        plsc.subcore_barrier()
        pltpu.sync_copy(acc_sh.at[pl.ds(s * s_per_t, s_per_t)],
                        rows_v.at[pl.ds(0, s_per_t)])
        pltpu.sync_copy(rows_v.at[pl.ds(0, s_per_t)],
                        out_hbm.at[c].at[pl.ds(s * s_per_t, s_per_t)])

    return scatter


def _dot(a, b):
    return jnp.dot(a, b, preferred_element_type=_f32)


def _edge_body(ea_ref, xg_ref, wa_ref, ba_ref, wb_ref, bb_ref, rep_ref,
               sel_ref, out_ref, *, aug, row_base):
    pid = pl.program_id(0)
    z = jnp.maximum(_dot(ea_ref[...], wa_ref[...]) + ba_ref[...], 0.0)
    h = _dot(z.astype(jnp.bfloat16), wb_ref[...].astype(jnp.bfloat16)) + bb_ref[...]
    xr = _dot(xg_ref[...], rep_ref[...])
    m = _dot(xr * h, sel_ref[...])
    if aug:
        col = lax.broadcasted_iota(jnp.int32, (EB, 16), 1)
        m = jnp.concatenate([m, jnp.where(col == 0, 1.0, 0.0)], axis=1)
    row = lax.broadcasted_iota(jnp.int32, m.shape, 0) + (row_base + pid * EB)
    out_ref[...] = jnp.where(row < N_EDGES, m, 0.0)


def _edge_call(ea, xg, wa, ba, wb, bb, rep, sel, out_w, aug, row_base):
    full_w = out_w + 16 if aug else out_w
    body = functools.partial(_edge_body, aug=aug, row_base=row_base)
    return pl.pallas_call(
        body,
        grid=(E_HALF // EB,),
        in_specs=[
            pl.BlockSpec((EB, D_EDGE), lambda i: (i, 0)),
            pl.BlockSpec((EB, xg.shape[1]), lambda i: (i, 0)),
            pl.BlockSpec(wa.shape, lambda i: (0, 0)),
            pl.BlockSpec(ba.shape, lambda i: (0, 0)),
            pl.BlockSpec(wb.shape, lambda i: (0, 0)),
            pl.BlockSpec(bb.shape, lambda i: (0, 0)),
            pl.BlockSpec(rep.shape, lambda i: (0, 0)),
            pl.BlockSpec(sel.shape, lambda i: (0, 0)),
        ],
        out_specs=pl.BlockSpec((EB, full_w), lambda i: (i, 0)),
        out_shape=jax.ShapeDtypeStruct((E_HALF, full_w), _f32),
    )(ea, xg, wa, ba, wb, bb, rep, sel)


def _node1_body(x_ref, pa_ref, pb_ref, root_ref, bias_ref, x1_ref, deg_ref):
    t = pa_ref[0] + pa_ref[1] + pb_ref[0] + pb_ref[1]
    agg = t[:, :HID]
    deg = t[:, HID:HID + 1]
    x1 = _dot(x_ref[...], root_ref[...]) + agg / jnp.maximum(deg, 1.0) + bias_ref[...]
    x1_ref[...] = jnp.maximum(x1, 0.0)
    deg_ref[...] = deg


def _node1_call(x, pa, pb, root, bias):
    return pl.pallas_call(
        _node1_body,
        grid=(N_PAD // NB,),
        in_specs=[
            pl.BlockSpec((NB, D_NODE), lambda i: (i, 0)),
            pl.BlockSpec((NC, NB, 2 * HID), lambda i: (0, i, 0)),
            pl.BlockSpec((NC, NB, 2 * HID), lambda i: (0, i, 0)),
            pl.BlockSpec(root.shape, lambda i: (0, 0)),
            pl.BlockSpec(bias.shape, lambda i: (0, 0)),
        ],
        out_specs=[
            pl.BlockSpec((NB, HID), lambda i: (i, 0)),
            pl.BlockSpec((NB, 1), lambda i: (i, 0)),
        ],
        out_shape=[
            jax.ShapeDtypeStruct((N_PAD, HID), _f32),
            jax.ShapeDtypeStruct((N_PAD, 1), _f32),
        ],
    )(x, pa, pb, root, bias)


def _node2_body(x1_ref, pa_ref, pb_ref, deg_ref, root_ref, bias_ref, out_ref):
    pid = pl.program_id(0)
    t = pa_ref[0] + pa_ref[1] + pb_ref[0] + pb_ref[1]
    x2 = _dot(x1_ref[...], root_ref[...]) + t / jnp.maximum(deg_ref[...], 1.0) + bias_ref[...]
    x2 = jnp.maximum(x2, 0.0)
    col = lax.broadcasted_iota(jnp.int32, (NB, 16), 1)
    aug = jnp.concatenate([x2, jnp.where(col == 0, 1.0, 0.0)], axis=1)
    row = lax.broadcasted_iota(jnp.int32, aug.shape, 0) + pid * NB
    out_ref[...] = jnp.where(row < N_NODES, aug, 0.0)


def _node2_call(x1, pa, pb, deg, root, bias):
    return pl.pallas_call(
        _node2_body,
        grid=(N_PAD // NB,),
        in_specs=[
            pl.BlockSpec((NB, HID), lambda i: (i, 0)),
            pl.BlockSpec((NC, NB, 2 * HID), lambda i: (0, i, 0)),
            pl.BlockSpec((NC, NB, 2 * HID), lambda i: (0, i, 0)),
            pl.BlockSpec((NB, 1), lambda i: (i, 0)),
            pl.BlockSpec(root.shape, lambda i: (0, 0)),
            pl.BlockSpec(bias.shape, lambda i: (0, 0)),
        ],
        out_specs=pl.BlockSpec((NB, 48), lambda i: (i, 0)),
        out_shape=jax.ShapeDtypeStruct((N_PAD, 48), _f32),
    )(x1, pa, pb, deg, root, bias)


def _head_body(p_ref, w1_ref, b1_ref, w2_ref, b2_ref, w3_ref, b3_ref,
               w4_ref, b4_ref, out_ref):
    t = p_ref[0] + p_ref[1]
    sums = t[:, :2 * HID]
    cnt = t[:, 2 * HID:2 * HID + 1]
    g = sums / jnp.maximum(cnt, 1.0)
    g = jnp.maximum(_dot(g, w1_ref[...]) + b1_ref[...], 0.0)
    g = jnp.maximum(_dot(g, w2_ref[...]) + b2_ref[...], 0.0)
    g = jnp.maximum(_dot(g, w3_ref[...]) + b3_ref[...], 0.0)
    out_ref[...] = _dot(g, w4_ref[...]) + b4_ref[...]


def _head_call(p, w1, b1, w2, b2, w3, b3, w4, b4):
    return pl.pallas_call(
        _head_body,
        out_shape=jax.ShapeDtypeStruct((N_GRAPHS, N_CLASSES), _f32),
    )(p, w1, b1, w2, b2, w3, b3, w4, b4)


def _rep_sel(in_ch, out_ch):
    k = in_ch * out_ch
    rep = np.zeros((in_ch, k), np.float32)
    rep[np.arange(k) // out_ch, np.arange(k)] = 1.0
    sel = np.zeros((k, out_ch), np.float32)
    sel[np.arange(k), np.arange(k) % out_ch] = 1.0
    return rep, sel


_REP1, _SEL1 = _rep_sel(D_NODE, HID)
_REP2, _SEL2 = _rep_sel(HID, 2 * HID)


def kernel(x, edge_index, edge_attr, batch, W1a, b1a, W1b, b1b, W2a, b2a,
           W2b, b2b, root1, bias1, root2, bias2, fcW1, fcb1, fcW2, fcb2,
           fcW3, fcb3, fcW4, fcb4):
    src = edge_index[0]
    dst = edge_index[1]
    ep = E_PAD - N_EDGES
    src_p = jnp.concatenate([src, jnp.zeros((ep,), jnp.int32)])
    dst_p = jnp.concatenate([dst, jnp.zeros((ep,), jnp.int32)])
    src3d = [src_p[h * E_HALF:(h + 1) * E_HALF].reshape(NW, -1, CH) for h in (0, 1)]
    dst3d = [dst_p[h * E_HALF:(h + 1) * E_HALF].reshape(NW, -1, CH) for h in (0, 1)]
    ea_p = jnp.concatenate([edge_attr, jnp.zeros((ep, D_EDGE), _f32)])
    ea_h = [ea_p[h * E_HALF:(h + 1) * E_HALF] for h in (0, 1)]
    npd = N_PAD - N_NODES
    x_p = jnp.concatenate([x, jnp.zeros((npd, D_NODE), _f32)])
    batch3d = jnp.concatenate([batch, jnp.zeros((npd,), jnp.int32)]).reshape(NW, -1, CH)

    z_node = jnp.zeros((N_PAD // NS, 2 * HID), _f32)
    z_pool = jnp.zeros((N_GRAPHS // NS, 48), _f32)

    b1a_, b1b_, b2a_, b2b_ = (b.reshape(1, -1) for b in (b1a, b1b, b2a, b2b))
    bias1_, bias2_ = bias1.reshape(1, -1), bias2.reshape(1, -1)
    fcb1_, fcb2_, fcb3_, fcb4_ = (b.reshape(1, -1) for b in (fcb1, fcb2, fcb3, fcb4))

    gather1 = _make_sc_gather(N_PAD, D_NODE, E_HALF)
    gather2 = _make_sc_gather(N_PAD, HID, E_HALF)
    scat_node = _make_sc_scatter(E_HALF, 2 * HID, N_PAD)

    xg1 = [gather1(x_p, s3) for s3 in src3d]
    msg1 = [_edge_call(ea_h[h], xg1[h], W1a, b1a_, W1b, b1b_, _REP1, _SEL1,
                       out_w=HID, aug=True, row_base=h * E_HALF) for h in (0, 1)]
    p1 = [scat_node(msg1[h], dst3d[h], z_node) for h in (0, 1)]
    x1, deg = _node1_call(x_p, p1[0], p1[1], root1, bias1_)

    xg2 = [gather2(x1, s3) for s3 in src3d]
    msg2 = [_edge_call(ea_h[h], xg2[h], W2a, b2a_, W2b, b2b_, _REP2, _SEL2,
                       out_w=2 * HID, aug=False, row_base=h * E_HALF) for h in (0, 1)]
    p2 = [scat_node(msg2[h], dst3d[h], z_node) for h in (0, 1)]
    h2 = _node2_call(x1, p2[0], p2[1], deg, root2, bias2_)

    gp = _make_sc_scatter(N_PAD, 48, N_GRAPHS)(h2, batch3d, z_pool)
    return _head_call(gp, fcW1, fcb1_, fcW2, fcb2_, fcW3, fcb3_, fcW4, fcb4_)


# speedup vs baseline: 2.4818x; 1.0223x over previous
import functools

import jax
import jax.numpy as jnp
import numpy as np
from jax import lax
from jax.experimental import pallas as pl
from jax.experimental.pallas import tpu as pltpu
from jax.experimental.pallas import tpu_sc as plsc

N_NODES = 20000
N_EDGES = 80000
N_GRAPHS = 512
D_NODE = 32
D_EDGE = 16
HID = 16
N_CLASSES = 10

NC = 2
NS = 16
NW = NC * NS
CH = 128

E_PAD = 81920
E_HALF = E_PAD // 2
N_PAD = 20480

EB = 2048
NB = 2048

_f32 = jnp.float32


def _mesh():
    return plsc.VectorSubcoreMesh(core_axis_name="c", subcore_axis_name="s")


_SC_PARAMS = pltpu.CompilerParams(use_tc_tiling_on_sc=False)


def _make_sc_gather(n_rows, d, n_idx, dtype=jnp.bfloat16):
    b_per_w = n_idx // NW
    n_ch = b_per_w // CH

    @functools.partial(
        pl.kernel,
        out_type=jax.ShapeDtypeStruct((n_idx, d), dtype),
        mesh=_mesh(),
        compiler_params=_SC_PARAMS,
        scratch_types=[
            pltpu.VMEM((n_ch, CH), jnp.int32),
            pltpu.VMEM((b_per_w, d), dtype),
            pltpu.SemaphoreType.DMA,
        ],
    )
    def gather(table_hbm, idx_hbm, out_hbm, idx_v, rows_v, sem):
        wid = lax.axis_index("c") * NS + lax.axis_index("s")
        pltpu.sync_copy(idx_hbm.at[wid], idx_v)
        cps = [
            pltpu.async_copy(table_hbm.at[idx_v.at[j]],
                             rows_v.at[pl.ds(j * CH, CH)], sem)
            for j in range(n_ch)
        ]
        for cp in cps:
            cp.wait()
        pltpu.sync_copy(rows_v, out_hbm.at[pl.ds(wid * b_per_w, b_per_w)])

    return gather


def _make_sc_scatter(n_rows, w, n_seg):
    b_per_w = n_rows // NW
    n_ch = b_per_w // CH
    s_per_t = n_seg // NS

    @functools.partial(
        pl.kernel,
        out_type=jax.ShapeDtypeStruct((NC, n_seg, w), _f32),
        mesh=_mesh(),
        compiler_params=_SC_PARAMS,
        scratch_types=[
            pltpu.VMEM((n_ch, CH), jnp.int32),
            pltpu.VMEM((b_per_w, w), _f32),
            pltpu.VMEM_SHARED((n_seg, w), _f32),
            pltpu.SemaphoreType.DMA,
        ],
    )
    def scatter(data_hbm, idx_hbm, zeros_hbm, out_hbm, idx_v, rows_v, acc_sh, sem):
        c = lax.axis_index("c")
        s = lax.axis_index("s")
        wid = c * NS + s
        pltpu.sync_copy(zeros_hbm, rows_v.at[pl.ds(0, s_per_t)])
        pltpu.sync_copy(rows_v.at[pl.ds(0, s_per_t)],
                        acc_sh.at[pl.ds(s * s_per_t, s_per_t)])
        plsc.subcore_barrier()
        pltpu.sync_copy(idx_hbm.at[wid], idx_v)
        pltpu.sync_copy(data_hbm.at[pl.ds(wid * b_per_w, b_per_w)], rows_v)
        for j in range(n_ch):
            pltpu.sync_copy(rows_v.at[pl.ds(j * CH, CH)],
                            acc_sh.at[idx_v.at[j]], add=True)
        plsc.subcore_barrier()
        pltpu.sync_copy(acc_sh.at[pl.ds(s * s_per_t, s_per_t)],
                        rows_v.at[pl.ds(0, s_per_t)])
        pltpu.sync_copy(rows_v.at[pl.ds(0, s_per_t)],
                        out_hbm.at[c].at[pl.ds(s * s_per_t, s_per_t)])

    return scatter


def _dot(a, b):
    return jnp.dot(a, b, preferred_element_type=_f32)


def _edge_body(ea_ref, xg_ref, wa_ref, ba_ref, wb_ref, bb_ref, rep_ref,
               sel_ref, out_ref, *, aug, row_base):
    pid = pl.program_id(0)
    z = jnp.maximum(_dot(ea_ref[...], wa_ref[...]) + ba_ref[...], 0.0)
    h = _dot(z.astype(jnp.bfloat16), wb_ref[...]) + bb_ref[...]
    xr = _dot(xg_ref[...], rep_ref[...])
    m = _dot(xr * h, sel_ref[...])
    if aug:
        col = lax.broadcasted_iota(jnp.int32, (EB, 16), 1)
        m = jnp.concatenate([m, jnp.where(col == 0, 1.0, 0.0)], axis=1)
    row = lax.broadcasted_iota(jnp.int32, m.shape, 0) + (row_base + pid * EB)
    out_ref[...] = jnp.where(row < N_EDGES, m, 0.0)


def _edge_call(ea, xg, wa, ba, wb, bb, rep, sel, out_w, aug, row_base):
    full_w = out_w + 16 if aug else out_w
    body = functools.partial(_edge_body, aug=aug, row_base=row_base)
    return pl.pallas_call(
        body,
        grid=(E_HALF // EB,),
        in_specs=[
            pl.BlockSpec((EB, D_EDGE), lambda i: (i, 0)),
            pl.BlockSpec((EB, xg.shape[1]), lambda i: (i, 0)),
            pl.BlockSpec(wa.shape, lambda i: (0, 0)),
            pl.BlockSpec(ba.shape, lambda i: (0, 0)),
            pl.BlockSpec(wb.shape, lambda i: (0, 0)),
            pl.BlockSpec(bb.shape, lambda i: (0, 0)),
            pl.BlockSpec(rep.shape, lambda i: (0, 0)),
            pl.BlockSpec(sel.shape, lambda i: (0, 0)),
        ],
        out_specs=pl.BlockSpec((EB, full_w), lambda i: (i, 0)),
        out_shape=jax.ShapeDtypeStruct((E_HALF, full_w), _f32),
    )(ea, xg, wa, ba, wb, bb, rep, sel)


def _node1_body(x_ref, pa_ref, pb_ref, root_ref, bias_ref, x1_ref, x1b_ref,
                deg_ref):
    t = pa_ref[0] + pa_ref[1] + pb_ref[0] + pb_ref[1]
    agg = t[:, :HID]
    deg = t[:, HID:HID + 1]
    x1 = _dot(x_ref[...], root_ref[...]) + agg / jnp.maximum(deg, 1.0) + bias_ref[...]
    x1 = jnp.maximum(x1, 0.0)
    x1_ref[...] = x1
    x1b_ref[...] = jnp.concatenate(
        [x1, jnp.zeros((NB, HID), _f32)], axis=1).astype(jnp.bfloat16)
    deg_ref[...] = deg


def _node1_call(x, pa, pb, root, bias):
    return pl.pallas_call(
        _node1_body,
        grid=(N_PAD // NB,),
        in_specs=[
            pl.BlockSpec((NB, D_NODE), lambda i: (i, 0)),
            pl.BlockSpec((NC, NB, 2 * HID), lambda i: (0, i, 0)),
            pl.BlockSpec((NC, NB, 2 * HID), lambda i: (0, i, 0)),
            pl.BlockSpec(root.shape, lambda i: (0, 0)),
            pl.BlockSpec(bias.shape, lambda i: (0, 0)),
        ],
        out_specs=[
            pl.BlockSpec((NB, HID), lambda i: (i, 0)),
            pl.BlockSpec((NB, 2 * HID), lambda i: (i, 0)),
            pl.BlockSpec((NB, 1), lambda i: (i, 0)),
        ],
        out_shape=[
            jax.ShapeDtypeStruct((N_PAD, HID), _f32),
            jax.ShapeDtypeStruct((N_PAD, 2 * HID), jnp.bfloat16),
            jax.ShapeDtypeStruct((N_PAD, 1), _f32),
        ],
    )(x, pa, pb, root, bias)


def _node2_body(x1_ref, pa_ref, pb_ref, deg_ref, root_ref, bias_ref, out_ref):
    pid = pl.program_id(0)
    t = pa_ref[0] + pa_ref[1] + pb_ref[0] + pb_ref[1]
    x2 = _dot(x1_ref[...], root_ref[...]) + t / jnp.maximum(deg_ref[...], 1.0) + bias_ref[...]
    x2 = jnp.maximum(x2, 0.0)
    col = lax.broadcasted_iota(jnp.int32, (NB, 16), 1)
    aug = jnp.concatenate([x2, jnp.where(col == 0, 1.0, 0.0)], axis=1)
    row = lax.broadcasted_iota(jnp.int32, aug.shape, 0) + pid * NB
    out_ref[...] = jnp.where(row < N_NODES, aug, 0.0)


def _node2_call(x1, pa, pb, deg, root, bias):
    return pl.pallas_call(
        _node2_body,
        grid=(N_PAD // NB,),
        in_specs=[
            pl.BlockSpec((NB, HID), lambda i: (i, 0)),
            pl.BlockSpec((NC, NB, 2 * HID), lambda i: (0, i, 0)),
            pl.BlockSpec((NC, NB, 2 * HID), lambda i: (0, i, 0)),
            pl.BlockSpec((NB, 1), lambda i: (i, 0)),
            pl.BlockSpec(root.shape, lambda i: (0, 0)),
            pl.BlockSpec(bias.shape, lambda i: (0, 0)),
        ],
        out_specs=pl.BlockSpec((NB, 48), lambda i: (i, 0)),
        out_shape=jax.ShapeDtypeStruct((N_PAD, 48), _f32),
    )(x1, pa, pb, deg, root, bias)


def _head_body(p_ref, w1_ref, b1_ref, w2_ref, b2_ref, w3_ref, b3_ref,
               w4_ref, b4_ref, out_ref):
    t = p_ref[0] + p_ref[1]
    sums = t[:, :2 * HID]
    cnt = t[:, 2 * HID:2 * HID + 1]
    g = sums / jnp.maximum(cnt, 1.0)
    g = jnp.maximum(_dot(g, w1_ref[...]) + b1_ref[...], 0.0)
    g = jnp.maximum(_dot(g, w2_ref[...]) + b2_ref[...], 0.0)
    g = jnp.maximum(_dot(g, w3_ref[...]) + b3_ref[...], 0.0)
    out_ref[...] = _dot(g, w4_ref[...]) + b4_ref[...]


def _head_call(p, w1, b1, w2, b2, w3, b3, w4, b4):
    return pl.pallas_call(
        _head_body,
        out_shape=jax.ShapeDtypeStruct((N_GRAPHS, N_CLASSES), _f32),
    )(p, w1, b1, w2, b2, w3, b3, w4, b4)


def _rep_sel(in_ch, out_ch):
    k = in_ch * out_ch
    rep = np.zeros((in_ch, k), np.float32)
    rep[np.arange(k) // out_ch, np.arange(k)] = 1.0
    sel = np.zeros((k, out_ch), np.float32)
    sel[np.arange(k), np.arange(k) % out_ch] = 1.0
    return rep, sel


_REP1, _SEL1 = _rep_sel(D_NODE, HID)
_REP2, _SEL2 = _rep_sel(HID, 2 * HID)
_REP2 = np.concatenate([_REP2, np.zeros((HID, HID * D_NODE), np.float32)], axis=0)


def kernel(x, edge_index, edge_attr, batch, W1a, b1a, W1b, b1b, W2a, b2a,
           W2b, b2b, root1, bias1, root2, bias2, fcW1, fcb1, fcW2, fcb2,
           fcW3, fcb3, fcW4, fcb4):
    src = edge_index[0]
    dst = edge_index[1]
    ep = E_PAD - N_EDGES
    src_p = jnp.concatenate([src, jnp.zeros((ep,), jnp.int32)])
    dst_p = jnp.concatenate([dst, jnp.zeros((ep,), jnp.int32)])
    src3d = [src_p[h * E_HALF:(h + 1) * E_HALF].reshape(NW, -1, CH) for h in (0, 1)]
    dst3d = [dst_p[h * E_HALF:(h + 1) * E_HALF].reshape(NW, -1, CH) for h in (0, 1)]
    ea_p = jnp.concatenate([edge_attr, jnp.zeros((ep, D_EDGE), _f32)])
    ea_h = [ea_p[h * E_HALF:(h + 1) * E_HALF] for h in (0, 1)]
    npd = N_PAD - N_NODES
    x_p = jnp.concatenate([x, jnp.zeros((npd, D_NODE), _f32)])
    batch3d = jnp.concatenate([batch, jnp.zeros((npd,), jnp.int32)]).reshape(NW, -1, CH)

    z_node = jnp.zeros((N_PAD // NS, 2 * HID), _f32)
    z_pool = jnp.zeros((N_GRAPHS // NS, 48), _f32)

    b1a_, b1b_, b2a_, b2b_ = (b.reshape(1, -1) for b in (b1a, b1b, b2a, b2b))
    bias1_, bias2_ = bias1.reshape(1, -1), bias2.reshape(1, -1)
    fcb1_, fcb2_, fcb3_, fcb4_ = (b.reshape(1, -1) for b in (fcb1, fcb2, fcb3, fcb4))

    gather1 = _make_sc_gather(N_PAD, D_NODE, E_HALF)
    gather2 = _make_sc_gather(N_PAD, 2 * HID, E_HALF)
    scat_node = _make_sc_scatter(E_HALF, 2 * HID, N_PAD)

    bf16 = jnp.bfloat16
    x_pb = x_p.astype(bf16)
    w1b_b = W1b.astype(bf16)
    w2b_b = W2b.astype(bf16)
    rep1_b = jnp.asarray(_REP1, bf16)
    rep2_b = jnp.asarray(_REP2, bf16)

    xg1 = [gather1(x_pb, s3) for s3 in src3d]
    msg1 = [_edge_call(ea_h[h], xg1[h], W1a, b1a_, w1b_b, b1b_, rep1_b, _SEL1,
                       out_w=HID, aug=True, row_base=h * E_HALF) for h in (0, 1)]
    p1 = [scat_node(msg1[h], dst3d[h], z_node) for h in (0, 1)]
    x1, x1b, deg = _node1_call(x_p, p1[0], p1[1], root1, bias1_)

    xg2 = [gather2(x1b, s3) for s3 in src3d]
    msg2 = [_edge_call(ea_h[h], xg2[h], W2a, b2a_, w2b_b, b2b_, rep2_b, _SEL2,
                       out_w=2 * HID, aug=False, row_base=h * E_HALF) for h in (0, 1)]
    p2 = [scat_node(msg2[h], dst3d[h], z_node) for h in (0, 1)]
    h2 = _node2_call(x1, p2[0], p2[1], deg, root2, bias2_)

    gp = _make_sc_scatter(N_PAD, 48, N_GRAPHS)(h2, batch3d, z_pool)
    return _head_call(gp, fcW1, fcb1_, fcW2, fcb2_, fcW3, fcb3_, fcW4, fcb4_)


# speedup vs baseline: 2.6173x; 1.0546x over previous
import functools

import jax
import jax.numpy as jnp
import numpy as np
from jax import lax
from jax.experimental import pallas as pl
from jax.experimental.pallas import tpu as pltpu
from jax.experimental.pallas import tpu_sc as plsc

N_NODES = 20000
N_EDGES = 80000
N_GRAPHS = 512
D_NODE = 32
D_EDGE = 16
HID = 16
N_CLASSES = 10

NC = 2
NS = 16
NW = NC * NS
CH = 128

E_PAD = 81920
E_HALF = E_PAD // 2
N_PAD = 20480
N_PER_T = N_PAD // NW
NSUB = 2
N_SUB = N_PER_T // NSUB

EB = 2048

_f32 = jnp.float32


def _mesh():
    return plsc.VectorSubcoreMesh(core_axis_name="c", subcore_axis_name="s")


_SC_PARAMS = pltpu.CompilerParams(use_tc_tiling_on_sc=False)


def _make_sc_gather(n_rows, d, n_idx, dtype=jnp.bfloat16):
    b_per_w = n_idx // NW
    n_ch = b_per_w // CH

    @functools.partial(
        pl.kernel,
        out_type=jax.ShapeDtypeStruct((n_idx, d), dtype),
        mesh=_mesh(),
        compiler_params=_SC_PARAMS,
        scratch_types=[
            pltpu.VMEM((n_ch, CH), jnp.int32),
            pltpu.VMEM((b_per_w, d), dtype),
            pltpu.SemaphoreType.DMA,
        ],
    )
    def gather(table_hbm, idx_hbm, out_hbm, idx_v, rows_v, sem):
        wid = lax.axis_index("c") * NS + lax.axis_index("s")
        pltpu.sync_copy(idx_hbm.at[wid], idx_v)
        cps = [
            pltpu.async_copy(table_hbm.at[idx_v.at[j]],
                             rows_v.at[pl.ds(j * CH, CH)], sem)
            for j in range(n_ch)
        ]
        for cp in cps:
            cp.wait()
        pltpu.sync_copy(rows_v, out_hbm.at[pl.ds(wid * b_per_w, b_per_w)])

    return gather


def _make_sc_scatter(n_rows, w, n_seg):
    b_per_w = n_rows // NW
    n_ch = b_per_w // CH
    s_per_t = n_seg // NS

    @functools.partial(
        pl.kernel,
        out_type=jax.ShapeDtypeStruct((NC, n_seg, w), _f32),
        mesh=_mesh(),
        compiler_params=_SC_PARAMS,
        scratch_types=[
            pltpu.VMEM((n_ch, CH), jnp.int32),
            pltpu.VMEM((b_per_w, w), _f32),
            pltpu.VMEM_SHARED((n_seg, w), _f32),
            pltpu.SemaphoreType.DMA,
        ],
    )
    def scatter(data_hbm, idx_hbm, zeros_hbm, out_hbm, idx_v, rows_v, acc_sh, sem):
        c = lax.axis_index("c")
        s = lax.axis_index("s")
        wid = c * NS + s
        pltpu.sync_copy(zeros_hbm, rows_v.at[pl.ds(0, s_per_t)])
        pltpu.sync_copy(rows_v.at[pl.ds(0, s_per_t)],
                        acc_sh.at[pl.ds(s * s_per_t, s_per_t)])
        plsc.subcore_barrier()
        pltpu.sync_copy(idx_hbm.at[wid], idx_v)
        pltpu.sync_copy(data_hbm.at[pl.ds(wid * b_per_w, b_per_w)], rows_v)
        for j in range(n_ch):
            pltpu.sync_copy(rows_v.at[pl.ds(j * CH, CH)],
                            acc_sh.at[idx_v.at[j]], add=True)
        plsc.subcore_barrier()
        pltpu.sync_copy(acc_sh.at[pl.ds(s * s_per_t, s_per_t)],
                        rows_v.at[pl.ds(0, s_per_t)])
        pltpu.sync_copy(rows_v.at[pl.ds(0, s_per_t)],
                        out_hbm.at[c].at[pl.ds(s * s_per_t, s_per_t)])

    return scatter


@functools.partial(
    pl.kernel,
    out_type=(jax.ShapeDtypeStruct((N_PAD, HID), _f32),
              jax.ShapeDtypeStruct((N_PAD, HID), _f32)),
    mesh=_mesh(),
    compiler_params=_SC_PARAMS,
    scratch_types=[
        pltpu.VMEM((N_SUB, 2 * HID), _f32),
        pltpu.VMEM((N_SUB, 2 * HID), _f32),
        pltpu.VMEM((N_SUB, 2 * HID), _f32),
        pltpu.VMEM((N_SUB, 2 * HID), _f32),
        pltpu.VMEM((N_SUB, HID), _f32),
        pltpu.VMEM((N_SUB, HID), _f32),
        pltpu.VMEM((N_SUB, HID), _f32),
        pltpu.SemaphoreType.DMA,
    ],
)
def _sc_node1(pa, pb, pre, x1_out, deg_out, q0, q1, q2, q3, pre_v, x1_v,
              deg_v, sem):
    wid = lax.axis_index("c") * NS + lax.axis_index("s")
    for sub in range(NSUB):
        base = wid * N_PER_T + sub * N_SUB
        pltpu.sync_copy(pa.at[0, pl.ds(base, N_SUB)], q0)
        pltpu.sync_copy(pa.at[1, pl.ds(base, N_SUB)], q1)
        pltpu.sync_copy(pb.at[0, pl.ds(base, N_SUB)], q2)
        pltpu.sync_copy(pb.at[1, pl.ds(base, N_SUB)], q3)
        pltpu.sync_copy(pre.at[pl.ds(base, N_SUB)], pre_v)

        def body(i, _):
            agg = (q0[i, pl.ds(0, HID)] + q1[i, pl.ds(0, HID)]
                   + q2[i, pl.ds(0, HID)] + q3[i, pl.ds(0, HID)])
            deg = (q0[i, pl.ds(HID, HID)] + q1[i, pl.ds(HID, HID)]
                   + q2[i, pl.ds(HID, HID)] + q3[i, pl.ds(HID, HID)])
            x1_v[i] = jnp.maximum(pre_v[i] + agg / jnp.maximum(deg, 1.0), 0.0)
            deg_v[i] = deg
            return 0

        lax.fori_loop(0, N_SUB, body, 0)
        pltpu.sync_copy(x1_v, x1_out.at[pl.ds(base, N_SUB)])
        pltpu.sync_copy(deg_v, deg_out.at[pl.ds(base, N_SUB)])


@functools.partial(
    pl.kernel,
    out_type=jax.ShapeDtypeStruct((NC, N_GRAPHS, 48), _f32),
    mesh=_mesh(),
    compiler_params=_SC_PARAMS,
    scratch_types=[
        pltpu.VMEM((N_SUB, 2 * HID), _f32),
        pltpu.VMEM((N_SUB, 2 * HID), _f32),
        pltpu.VMEM((N_SUB, 2 * HID), _f32),
        pltpu.VMEM((N_SUB, 2 * HID), _f32),
        pltpu.VMEM((N_SUB, 2 * HID), _f32),
        pltpu.VMEM((N_SUB, HID), _f32),
        pltpu.VMEM((N_PER_T, 48), _f32),
        pltpu.VMEM((N_PER_T // CH, CH), jnp.int32),
        pltpu.VMEM_SHARED((N_GRAPHS, 48), _f32),
        pltpu.SemaphoreType.DMA,
    ],
)
def _sc_node2_pool(pa, pb, pre, degi, bidx, zeros_hbm, out, q0, q1, q2, q3,
                   pre_v, deg_v, haug, idx_v, acc_sh, sem):
    c = lax.axis_index("c")
    s = lax.axis_index("s")
    wid = c * NS + s
    spt = N_GRAPHS // NS
    pltpu.sync_copy(zeros_hbm, haug.at[pl.ds(0, spt)])
    pltpu.sync_copy(haug.at[pl.ds(0, spt)], acc_sh.at[pl.ds(s * spt, spt)])
    plsc.subcore_barrier()
    pltpu.sync_copy(bidx.at[wid], idx_v)
    for sub in range(NSUB):
        base = wid * N_PER_T + sub * N_SUB
        pltpu.sync_copy(pa.at[0, pl.ds(base, N_SUB)], q0)
        pltpu.sync_copy(pa.at[1, pl.ds(base, N_SUB)], q1)
        pltpu.sync_copy(pb.at[0, pl.ds(base, N_SUB)], q2)
        pltpu.sync_copy(pb.at[1, pl.ds(base, N_SUB)], q3)
        pltpu.sync_copy(pre.at[pl.ds(base, N_SUB)], pre_v)
        pltpu.sync_copy(degi.at[pl.ds(base, N_SUB)], deg_v)

        def body(i, _):
            t0 = (q0[i, pl.ds(0, HID)] + q1[i, pl.ds(0, HID)]
                  + q2[i, pl.ds(0, HID)] + q3[i, pl.ds(0, HID)])
            t1 = (q0[i, pl.ds(HID, HID)] + q1[i, pl.ds(HID, HID)]
                  + q2[i, pl.ds(HID, HID)] + q3[i, pl.ds(HID, HID)])
            degc = jnp.maximum(deg_v[i], 1.0)
            h2a = jnp.maximum(pre_v[i, pl.ds(0, HID)] + t0 / degc, 0.0)
            h2b = jnp.maximum(pre_v[i, pl.ds(HID, HID)] + t1 / degc, 0.0)
            one = jnp.where(base + i < N_NODES, 1.0, 0.0)
            row = sub * N_SUB + i
            haug[row, pl.ds(0, HID)] = h2a * one
            haug[row, pl.ds(HID, HID)] = h2b * one
            haug[row, pl.ds(2 * HID, HID)] = jnp.full((HID,), 1.0, _f32) * one
            return 0

        lax.fori_loop(0, N_SUB, body, 0)
    for j in range(N_PER_T // CH):
        pltpu.sync_copy(haug.at[pl.ds(j * CH, CH)],
                        acc_sh.at[idx_v.at[j]], add=True)
    plsc.subcore_barrier()
    pltpu.sync_copy(acc_sh.at[pl.ds(s * spt, spt)], haug.at[pl.ds(0, spt)])
    pltpu.sync_copy(haug.at[pl.ds(0, spt)], out.at[c].at[pl.ds(s * spt, spt)])


def _dot(a, b):
    return jnp.dot(a, b, preferred_element_type=_f32)


def _edge_body(ea_ref, xg_ref, wa_ref, ba_ref, wb_ref, bb_ref, rep_ref,
               sel_ref, out_ref, *, aug, row_base):
    pid = pl.program_id(0)
    z = jnp.maximum(_dot(ea_ref[...], wa_ref[...]) + ba_ref[...], 0.0)
    h = _dot(z.astype(jnp.bfloat16), wb_ref[...]) + bb_ref[...]
    xr = _dot(xg_ref[...], rep_ref[...])
    m = _dot(xr * h, sel_ref[...])
    if aug:
        m = jnp.concatenate([m, jnp.ones((EB, HID), _f32)], axis=1)
    row = lax.broadcasted_iota(jnp.int32, m.shape, 0) + (row_base + pid * EB)
    out_ref[...] = jnp.where(row < N_EDGES, m, 0.0)


def _edge_call(ea, xg, wa, ba, wb, bb, rep, sel, out_w, aug, row_base):
    full_w = out_w + HID if aug else out_w
    body = functools.partial(_edge_body, aug=aug, row_base=row_base)
    return pl.pallas_call(
        body,
        grid=(E_HALF // EB,),
        in_specs=[
            pl.BlockSpec((EB, D_EDGE), lambda i: (i, 0)),
            pl.BlockSpec((EB, xg.shape[1]), lambda i: (i, 0)),
            pl.BlockSpec(wa.shape, lambda i: (0, 0)),
            pl.BlockSpec(ba.shape, lambda i: (0, 0)),
            pl.BlockSpec(wb.shape, lambda i: (0, 0)),
            pl.BlockSpec(bb.shape, lambda i: (0, 0)),
            pl.BlockSpec(rep.shape, lambda i: (0, 0)),
            pl.BlockSpec(sel.shape, lambda i: (0, 0)),
        ],
        out_specs=pl.BlockSpec((EB, full_w), lambda i: (i, 0)),
        out_shape=jax.ShapeDtypeStruct((E_HALF, full_w), _f32),
    )(ea, xg, wa, ba, wb, bb, rep, sel)


def _pre_body(x_ref, w_ref, b_ref, o_ref):
    o_ref[...] = _dot(x_ref[...], w_ref[...]) + b_ref[...]


def _pre_call(x, w, b):
    return pl.pallas_call(
        _pre_body,
        out_shape=jax.ShapeDtypeStruct((N_PAD, w.shape[1]), _f32),
    )(x, w, b)


def _head_body(p_ref, w1_ref, b1_ref, w2_ref, b2_ref, w3_ref, b3_ref,
               w4_ref, b4_ref, out_ref):
    t = p_ref[0] + p_ref[1]
    sums = t[:, :2 * HID]
    cnt = t[:, 2 * HID:2 * HID + 1]
    g = sums / jnp.maximum(cnt, 1.0)
    g = jnp.maximum(_dot(g, w1_ref[...]) + b1_ref[...], 0.0)
    g = jnp.maximum(_dot(g, w2_ref[...]) + b2_ref[...], 0.0)
    g = jnp.maximum(_dot(g, w3_ref[...]) + b3_ref[...], 0.0)
    out_ref[...] = _dot(g, w4_ref[...]) + b4_ref[...]


def _head_call(p, w1, b1, w2, b2, w3, b3, w4, b4):
    return pl.pallas_call(
        _head_body,
        out_shape=jax.ShapeDtypeStruct((N_GRAPHS, N_CLASSES), _f32),
    )(p, w1, b1, w2, b2, w3, b3, w4, b4)


def _rep_sel(in_ch, out_ch):
    k = in_ch * out_ch
    rep = np.zeros((in_ch, k), np.float32)
    rep[np.arange(k) // out_ch, np.arange(k)] = 1.0
    sel = np.zeros((k, out_ch), np.float32)
    sel[np.arange(k), np.arange(k) % out_ch] = 1.0
    return rep, sel


_REP1, _SEL1 = _rep_sel(D_NODE, HID)
_REP2, _SEL2 = _rep_sel(HID, 2 * HID)


def kernel(x, edge_index, edge_attr, batch, W1a, b1a, W1b, b1b, W2a, b2a,
           W2b, b2b, root1, bias1, root2, bias2, fcW1, fcb1, fcW2, fcb2,
           fcW3, fcb3, fcW4, fcb4):
    src = edge_index[0]
    dst = edge_index[1]
    ep = E_PAD - N_EDGES
    src_p = jnp.concatenate([src, jnp.zeros((ep,), jnp.int32)])
    dst_p = jnp.concatenate([dst, jnp.zeros((ep,), jnp.int32)])
    src3d = [src_p[h * E_HALF:(h + 1) * E_HALF].reshape(NW, -1, CH) for h in (0, 1)]
    dst3d = [dst_p[h * E_HALF:(h + 1) * E_HALF].reshape(NW, -1, CH) for h in (0, 1)]
    ea_p = jnp.concatenate([edge_attr, jnp.zeros((ep, D_EDGE), _f32)])
    ea_h = [ea_p[h * E_HALF:(h + 1) * E_HALF] for h in (0, 1)]
    npd = N_PAD - N_NODES
    x_p = jnp.concatenate([x, jnp.zeros((npd, D_NODE), _f32)])
    batch3d = jnp.concatenate([batch, jnp.zeros((npd,), jnp.int32)]).reshape(NW, -1, CH)

    z_node = jnp.zeros((N_PAD // NS, 2 * HID), _f32)
    z_pool = jnp.zeros((N_GRAPHS // NS, 48), _f32)

    b1a_, b1b_, b2a_, b2b_ = (b.reshape(1, -1) for b in (b1a, b1b, b2a, b2b))
    fcb1_, fcb2_, fcb3_, fcb4_ = (b.reshape(1, -1) for b in (fcb1, fcb2, fcb3, fcb4))

    gather1 = _make_sc_gather(N_PAD, D_NODE, E_HALF)
    gather2 = _make_sc_gather(N_PAD, HID, E_HALF, dtype=_f32)
    scat_node = _make_sc_scatter(E_HALF, 2 * HID, N_PAD)

    bf16 = jnp.bfloat16
    x_pb = x_p.astype(bf16)
    w1b_b = W1b.astype(bf16)
    w2b_b = W2b.astype(bf16)
    rep1_b = jnp.asarray(_REP1, bf16)

    pre1 = _pre_call(x_p, root1, bias1.reshape(1, -1))
    xg1 = [gather1(x_pb, s3) for s3 in src3d]
    msg1 = [_edge_call(ea_h[h], xg1[h], W1a, b1a_, w1b_b, b1b_, rep1_b, _SEL1,
                       out_w=HID, aug=True, row_base=h * E_HALF) for h in (0, 1)]
    p1 = [scat_node(msg1[h], dst3d[h], z_node) for h in (0, 1)]
    x1, degv = _sc_node1(p1[0], p1[1], pre1)

    pre2 = _pre_call(x1, root2, bias2.reshape(1, -1))
    xg2 = [gather2(x1, s3) for s3 in src3d]
    msg2 = [_edge_call(ea_h[h], xg2[h], W2a, b2a_, w2b_b, b2b_, _REP2, _SEL2,
                       out_w=2 * HID, aug=False, row_base=h * E_HALF) for h in (0, 1)]
    p2 = [scat_node(msg2[h], dst3d[h], z_node) for h in (0, 1)]

    gp = _sc_node2_pool(p2[0], p2[1], pre2, degv, batch3d, z_pool)
    return _head_call(gp, fcW1, fcb1_, fcW2, fcb2_, fcW3, fcb3_, fcW4, fcb4_)


# speedup vs baseline: 2.6205x; 1.0012x over previous
import functools

import jax
import jax.numpy as jnp
import numpy as np
from jax import lax
from jax.experimental import pallas as pl
from jax.experimental.pallas import tpu as pltpu
from jax.experimental.pallas import tpu_sc as plsc

N_NODES = 20000
N_EDGES = 80000
N_GRAPHS = 512
D_NODE = 32
D_EDGE = 16
HID = 16
N_CLASSES = 10

NC = 2
NS = 16
NW = NC * NS
CH = 128

E_PAD = 81920
E_HALF = E_PAD // 2
N_PAD = 20480
N_PER_T = N_PAD // NW
NSUB = 2
N_SUB = N_PER_T // NSUB

EB = 2048

_f32 = jnp.float32


def _mesh():
    return plsc.VectorSubcoreMesh(core_axis_name="c", subcore_axis_name="s")


_SC_PARAMS = pltpu.CompilerParams(use_tc_tiling_on_sc=False)


def _make_sc_gather(n_rows, d, n_idx, dtype=jnp.bfloat16):
    b_per_w = n_idx // NW
    n_ch = b_per_w // CH

    @functools.partial(
        pl.kernel,
        out_type=jax.ShapeDtypeStruct((n_idx, d), dtype),
        mesh=_mesh(),
        compiler_params=_SC_PARAMS,
        scratch_types=[
            pltpu.VMEM((n_ch, CH), jnp.int32),
            pltpu.VMEM((b_per_w, d), dtype),
            pltpu.SemaphoreType.DMA,
        ],
    )
    def gather(table_hbm, idx_hbm, out_hbm, idx_v, rows_v, sem):
        wid = lax.axis_index("c") * NS + lax.axis_index("s")
        pltpu.sync_copy(idx_hbm.at[wid], idx_v)
        cps = [
            pltpu.async_copy(table_hbm.at[idx_v.at[j]],
                             rows_v.at[pl.ds(j * CH, CH)], sem)
            for j in range(n_ch)
        ]
        for cp in cps:
            cp.wait()
        pltpu.sync_copy(rows_v, out_hbm.at[pl.ds(wid * b_per_w, b_per_w)])

    return gather


def _make_sc_scatter(n_rows, w, n_seg):
    b_per_w = n_rows // NW
    n_ch = b_per_w // CH
    s_per_t = n_seg // NS

    @functools.partial(
        pl.kernel,
        out_type=jax.ShapeDtypeStruct((NC, n_seg, w), _f32),
        mesh=_mesh(),
        compiler_params=_SC_PARAMS,
        scratch_types=[
            pltpu.VMEM((n_ch, CH), jnp.int32),
            pltpu.VMEM((b_per_w, w), _f32),
            pltpu.VMEM_SHARED((n_seg, w), _f32),
            pltpu.SemaphoreType.DMA,
        ],
    )
    def scatter(data_hbm, idx_hbm, zeros_hbm, out_hbm, idx_v, rows_v, acc_sh, sem):
        c = lax.axis_index("c")
        s = lax.axis_index("s")
        wid = c * NS + s
        pltpu.sync_copy(zeros_hbm, rows_v.at[pl.ds(0, s_per_t)])
        pltpu.sync_copy(rows_v.at[pl.ds(0, s_per_t)],
                        acc_sh.at[pl.ds(s * s_per_t, s_per_t)])
        plsc.subcore_barrier()
        pltpu.sync_copy(idx_hbm.at[wid], idx_v)
        pltpu.sync_copy(data_hbm.at[pl.ds(wid * b_per_w, b_per_w)], rows_v)
        for j in range(n_ch):
            pltpu.sync_copy(rows_v.at[pl.ds(j * CH, CH)],
                            acc_sh.at[idx_v.at[j]], add=True)
        plsc.subcore_barrier()
        pltpu.sync_copy(acc_sh.at[pl.ds(s * s_per_t, s_per_t)],
                        rows_v.at[pl.ds(0, s_per_t)])
        pltpu.sync_copy(rows_v.at[pl.ds(0, s_per_t)],
                        out_hbm.at[c].at[pl.ds(s * s_per_t, s_per_t)])

    return scatter


@functools.partial(
    pl.kernel,
    out_type=(jax.ShapeDtypeStruct((N_PAD, HID), _f32),
              jax.ShapeDtypeStruct((N_PAD, HID), _f32)),
    mesh=_mesh(),
    compiler_params=_SC_PARAMS,
    scratch_types=[
        pltpu.VMEM((N_SUB, 2 * HID), _f32),
        pltpu.VMEM((N_SUB, 2 * HID), _f32),
        pltpu.VMEM((N_SUB, 2 * HID), _f32),
        pltpu.VMEM((N_SUB, 2 * HID), _f32),
        pltpu.VMEM((N_SUB, HID), _f32),
        pltpu.VMEM((N_SUB, HID), _f32),
        pltpu.VMEM((N_SUB, HID), _f32),
        pltpu.SemaphoreType.DMA,
    ],
)
def _sc_node1(pa, pb, pre, x1_out, deg_out, q0, q1, q2, q3, pre_v, x1_v,
              deg_v, sem):
    wid = lax.axis_index("c") * NS + lax.axis_index("s")
    for sub in range(NSUB):
        base = wid * N_PER_T + sub * N_SUB
        pltpu.sync_copy(pa.at[0, pl.ds(base, N_SUB)], q0)
        pltpu.sync_copy(pa.at[1, pl.ds(base, N_SUB)], q1)
        pltpu.sync_copy(pb.at[0, pl.ds(base, N_SUB)], q2)
        pltpu.sync_copy(pb.at[1, pl.ds(base, N_SUB)], q3)
        pltpu.sync_copy(pre.at[pl.ds(base, N_SUB)], pre_v)

        def body(i, _):
            agg = (q0[i, pl.ds(0, HID)] + q1[i, pl.ds(0, HID)]
                   + q2[i, pl.ds(0, HID)] + q3[i, pl.ds(0, HID)])
            deg = (q0[i, pl.ds(HID, HID)] + q1[i, pl.ds(HID, HID)]
                   + q2[i, pl.ds(HID, HID)] + q3[i, pl.ds(HID, HID)])
            x1_v[i] = jnp.maximum(pre_v[i] + agg / jnp.maximum(deg, 1.0), 0.0)
            deg_v[i] = deg
            return 0

        lax.fori_loop(0, N_SUB, body, 0)
        pltpu.sync_copy(x1_v, x1_out.at[pl.ds(base, N_SUB)])
        pltpu.sync_copy(deg_v, deg_out.at[pl.ds(base, N_SUB)])


@functools.partial(
    pl.kernel,
    out_type=jax.ShapeDtypeStruct((NC, N_GRAPHS, 48), _f32),
    mesh=_mesh(),
    compiler_params=_SC_PARAMS,
    scratch_types=[
        pltpu.VMEM((N_SUB, 2 * HID), _f32),
        pltpu.VMEM((N_SUB, 2 * HID), _f32),
        pltpu.VMEM((N_SUB, 2 * HID), _f32),
        pltpu.VMEM((N_SUB, 2 * HID), _f32),
        pltpu.VMEM((N_SUB, 2 * HID), _f32),
        pltpu.VMEM((N_SUB, HID), _f32),
        pltpu.VMEM((N_PER_T, 48), _f32),
        pltpu.VMEM((N_PER_T // CH, CH), jnp.int32),
        pltpu.VMEM_SHARED((N_GRAPHS, 48), _f32),
        pltpu.SemaphoreType.DMA,
    ],
)
def _sc_node2_pool(pa, pb, pre, degi, bidx, zeros_hbm, out, q0, q1, q2, q3,
                   pre_v, deg_v, haug, idx_v, acc_sh, sem):
    c = lax.axis_index("c")
    s = lax.axis_index("s")
    wid = c * NS + s
    spt = N_GRAPHS // NS
    pltpu.sync_copy(zeros_hbm, haug.at[pl.ds(0, spt)])
    pltpu.sync_copy(haug.at[pl.ds(0, spt)], acc_sh.at[pl.ds(s * spt, spt)])
    plsc.subcore_barrier()
    pltpu.sync_copy(bidx.at[wid], idx_v)
    for sub in range(NSUB):
        base = wid * N_PER_T + sub * N_SUB
        pltpu.sync_copy(pa.at[0, pl.ds(base, N_SUB)], q0)
        pltpu.sync_copy(pa.at[1, pl.ds(base, N_SUB)], q1)
        pltpu.sync_copy(pb.at[0, pl.ds(base, N_SUB)], q2)
        pltpu.sync_copy(pb.at[1, pl.ds(base, N_SUB)], q3)
        pltpu.sync_copy(pre.at[pl.ds(base, N_SUB)], pre_v)
        pltpu.sync_copy(degi.at[pl.ds(base, N_SUB)], deg_v)

        def body(i, _):
            t0 = (q0[i, pl.ds(0, HID)] + q1[i, pl.ds(0, HID)]
                  + q2[i, pl.ds(0, HID)] + q3[i, pl.ds(0, HID)])
            t1 = (q0[i, pl.ds(HID, HID)] + q1[i, pl.ds(HID, HID)]
                  + q2[i, pl.ds(HID, HID)] + q3[i, pl.ds(HID, HID)])
            degc = jnp.maximum(deg_v[i], 1.0)
            h2a = jnp.maximum(pre_v[i, pl.ds(0, HID)] + t0 / degc, 0.0)
            h2b = jnp.maximum(pre_v[i, pl.ds(HID, HID)] + t1 / degc, 0.0)
            one = jnp.where(base + i < N_NODES, 1.0, 0.0)
            row = sub * N_SUB + i
            haug[row, pl.ds(0, HID)] = h2a * one
            haug[row, pl.ds(HID, HID)] = h2b * one
            haug[row, pl.ds(2 * HID, HID)] = jnp.full((HID,), 1.0, _f32) * one
            return 0

        lax.fori_loop(0, N_SUB, body, 0)
    for j in range(N_PER_T // CH):
        pltpu.sync_copy(haug.at[pl.ds(j * CH, CH)],
                        acc_sh.at[idx_v.at[j]], add=True)
    plsc.subcore_barrier()
    pltpu.sync_copy(acc_sh.at[pl.ds(s * spt, spt)], haug.at[pl.ds(0, spt)])
    pltpu.sync_copy(haug.at[pl.ds(0, spt)], out.at[c].at[pl.ds(s * spt, spt)])


def _dot(a, b):
    return jnp.dot(a, b, preferred_element_type=_f32)


def _edge_body(ea_ref, xg_ref, wa_ref, ba_ref, wb_ref, bb_ref, rep_ref,
               sel_ref, out_ref, *, aug, row_base):
    pid = pl.program_id(0)
    z = jnp.maximum(_dot(ea_ref[...], wa_ref[...]) + ba_ref[...], 0.0)
    h = _dot(z.astype(jnp.bfloat16), wb_ref[...]) + bb_ref[...]
    xr = _dot(xg_ref[...], rep_ref[...])
    m = _dot(xr * h, sel_ref[...])
    if aug:
        m = jnp.concatenate([m, jnp.ones((EB, HID), _f32)], axis=1)
    row = lax.broadcasted_iota(jnp.int32, m.shape, 0) + (row_base + pid * EB)
    out_ref[...] = jnp.where(row < N_EDGES, m, 0.0)


def _edge_call(ea, xg, wa, ba, wb, bb, rep, sel, out_w, aug, row_base):
    full_w = out_w + HID if aug else out_w
    body = functools.partial(_edge_body, aug=aug, row_base=row_base)
    blk_off = row_base // EB
    return pl.pallas_call(
        body,
        grid=(E_HALF // EB,),
        in_specs=[
            pl.BlockSpec((EB, D_EDGE), lambda i: (i + blk_off, 0)),
            pl.BlockSpec((EB, xg.shape[1]), lambda i: (i, 0)),
            pl.BlockSpec(wa.shape, lambda i: (0, 0)),
            pl.BlockSpec(ba.shape, lambda i: (0, 0)),
            pl.BlockSpec(wb.shape, lambda i: (0, 0)),
            pl.BlockSpec(bb.shape, lambda i: (0, 0)),
            pl.BlockSpec(rep.shape, lambda i: (0, 0)),
            pl.BlockSpec(sel.shape, lambda i: (0, 0)),
        ],
        out_specs=pl.BlockSpec((EB, full_w), lambda i: (i, 0)),
        out_shape=jax.ShapeDtypeStruct((E_HALF, full_w), _f32),
    )(ea, xg, wa, ba, wb, bb, rep, sel)


def _pre_body(x_ref, w_ref, b_ref, o_ref):
    o_ref[...] = _dot(x_ref[...], w_ref[...]) + b_ref[...]


def _pre_call(x, w, b):
    return pl.pallas_call(
        _pre_body,
        out_shape=jax.ShapeDtypeStruct((N_PAD, w.shape[1]), _f32),
    )(x, w, b)


def _head_body(p_ref, w1_ref, b1_ref, w2_ref, b2_ref, w3_ref, b3_ref,
               w4_ref, b4_ref, out_ref):
    t = p_ref[0] + p_ref[1]
    sums = t[:, :2 * HID]
    cnt = t[:, 2 * HID:2 * HID + 1]
    g = sums / jnp.maximum(cnt, 1.0)
    g = jnp.maximum(_dot(g, w1_ref[...]) + b1_ref[...], 0.0)
    g = jnp.maximum(_dot(g, w2_ref[...]) + b2_ref[...], 0.0)
    g = jnp.maximum(_dot(g, w3_ref[...]) + b3_ref[...], 0.0)
    out_ref[...] = _dot(g, w4_ref[...]) + b4_ref[...]


def _head_call(p, w1, b1, w2, b2, w3, b3, w4, b4):
    return pl.pallas_call(
        _head_body,
        out_shape=jax.ShapeDtypeStruct((N_GRAPHS, N_CLASSES), _f32),
    )(p, w1, b1, w2, b2, w3, b3, w4, b4)


def _rep_sel(in_ch, out_ch):
    k = in_ch * out_ch
    rep = np.zeros((in_ch, k), np.float32)
    rep[np.arange(k) // out_ch, np.arange(k)] = 1.0
    sel = np.zeros((k, out_ch), np.float32)
    sel[np.arange(k), np.arange(k) % out_ch] = 1.0
    return rep, sel


_REP1, _SEL1 = _rep_sel(D_NODE, HID)
_REP2, _SEL2 = _rep_sel(HID, 2 * HID)


def kernel(x, edge_index, edge_attr, batch, W1a, b1a, W1b, b1b, W2a, b2a,
           W2b, b2b, root1, bias1, root2, bias2, fcW1, fcb1, fcW2, fcb2,
           fcW3, fcb3, fcW4, fcb4):
    src = edge_index[0]
    dst = edge_index[1]
    ep = E_PAD - N_EDGES
    src_p = jnp.concatenate([src, jnp.zeros((ep,), jnp.int32)])
    dst_p = jnp.concatenate([dst, jnp.zeros((ep,), jnp.int32)])
    src3d = [src_p[h * E_HALF:(h + 1) * E_HALF].reshape(NW, -1, CH) for h in (0, 1)]
    dst3d = [dst_p[h * E_HALF:(h + 1) * E_HALF].reshape(NW, -1, CH) for h in (0, 1)]
    ea_p = jnp.concatenate([edge_attr, jnp.zeros((ep, D_EDGE), _f32)])
    npd = N_PAD - N_NODES
    x_p = jnp.concatenate([x, jnp.zeros((npd, D_NODE), _f32)])
    batch3d = jnp.concatenate([batch, jnp.zeros((npd,), jnp.int32)]).reshape(NW, -1, CH)

    z_node = jnp.zeros((N_PAD // NS, 2 * HID), _f32)
    z_pool = jnp.zeros((N_GRAPHS // NS, 48), _f32)

    b1a_, b1b_, b2a_, b2b_ = (b.reshape(1, -1) for b in (b1a, b1b, b2a, b2b))
    fcb1_, fcb2_, fcb3_, fcb4_ = (b.reshape(1, -1) for b in (fcb1, fcb2, fcb3, fcb4))

    gather1 = _make_sc_gather(N_PAD, D_NODE, E_HALF)
    gather2 = _make_sc_gather(N_PAD, HID, E_HALF, dtype=_f32)
    scat_node = _make_sc_scatter(E_HALF, 2 * HID, N_PAD)

    bf16 = jnp.bfloat16
    x_pb = x_p.astype(bf16)
    w1b_b = W1b.astype(bf16)
    w2b_b = W2b.astype(bf16)
    rep1_b = jnp.asarray(_REP1, bf16)

    pre1 = _pre_call(x_p, root1, bias1.reshape(1, -1))
    xg1 = [gather1(x_pb, s3) for s3 in src3d]
    msg1 = [_edge_call(ea_p, xg1[h], W1a, b1a_, w1b_b, b1b_, rep1_b, _SEL1,
                       out_w=HID, aug=True, row_base=h * E_HALF) for h in (0, 1)]
    p1 = [scat_node(msg1[h], dst3d[h], z_node) for h in (0, 1)]
    x1, degv = _sc_node1(p1[0], p1[1], pre1)

    pre2 = _pre_call(x1, root2, bias2.reshape(1, -1))
    xg2 = [gather2(x1, s3) for s3 in src3d]
    msg2 = [_edge_call(ea_p, xg2[h], W2a, b2a_, w2b_b, b2b_, _REP2, _SEL2,
                       out_w=2 * HID, aug=False, row_base=h * E_HALF) for h in (0, 1)]
    p2 = [scat_node(msg2[h], dst3d[h], z_node) for h in (0, 1)]

    gp = _sc_node2_pool(p2[0], p2[1], pre2, degv, batch3d, z_pool)
    return _head_call(gp, fcW1, fcb1_, fcW2, fcb2_, fcW3, fcb3_, fcW4, fcb4_)


# speedup vs baseline: 2.6719x; 1.0196x over previous
import functools

import jax
import jax.numpy as jnp
import numpy as np
from jax import lax
from jax.experimental import pallas as pl
from jax.experimental.pallas import tpu as pltpu
from jax.experimental.pallas import tpu_sc as plsc

N_NODES = 20000
N_EDGES = 80000
N_GRAPHS = 512
D_NODE = 32
D_EDGE = 16
HID = 16
N_CLASSES = 10

NC = 2
NS = 16
NW = NC * NS
CH = 128

E_PAD = 81920
E_HALF = E_PAD // 2
N_PAD = 20480
N_PER_T = N_PAD // NW
NSUB = 2
N_SUB = N_PER_T // NSUB

EB = 2048

_f32 = jnp.float32


def _mesh():
    return plsc.VectorSubcoreMesh(core_axis_name="c", subcore_axis_name="s")


_SC_PARAMS = pltpu.CompilerParams(use_tc_tiling_on_sc=False)


def _make_sc_gather(n_rows, d, n_idx, dtype=jnp.bfloat16):
    b_per_w = n_idx // NW
    n_ch = b_per_w // CH

    @functools.partial(
        pl.kernel,
        out_type=jax.ShapeDtypeStruct((n_idx, d), dtype),
        mesh=_mesh(),
        compiler_params=_SC_PARAMS,
        scratch_types=[
            pltpu.VMEM((n_ch, CH), jnp.int32),
            pltpu.VMEM((b_per_w, d), dtype),
            pltpu.SemaphoreType.DMA,
        ],
    )
    def gather(table_hbm, idx_hbm, out_hbm, idx_v, rows_v, sem):
        wid = lax.axis_index("c") * NS + lax.axis_index("s")
        pltpu.sync_copy(idx_hbm.at[wid], idx_v)
        cps = [
            pltpu.async_copy(table_hbm.at[idx_v.at[j]],
                             rows_v.at[pl.ds(j * CH, CH)], sem)
            for j in range(n_ch)
        ]
        for cp in cps:
            cp.wait()
        pltpu.sync_copy(rows_v, out_hbm.at[pl.ds(wid * b_per_w, b_per_w)])

    return gather


def _make_sc_scatter(n_rows, w, n_seg):
    b_per_w = n_rows // NW
    n_ch = b_per_w // CH
    s_per_t = n_seg // NS

    @functools.partial(
        pl.kernel,
        out_type=jax.ShapeDtypeStruct((NC, n_seg, w), _f32),
        mesh=_mesh(),
        compiler_params=_SC_PARAMS,
        scratch_types=[
            pltpu.VMEM((n_ch, CH), jnp.int32),
            pltpu.VMEM((b_per_w, w), _f32),
            pltpu.VMEM_SHARED((n_seg, w), _f32),
            pltpu.SemaphoreType.DMA,
        ],
    )
    def scatter(data_hbm, idx_hbm, zeros_hbm, out_hbm, idx_v, rows_v, acc_sh, sem):
        c = lax.axis_index("c")
        s = lax.axis_index("s")
        wid = c * NS + s
        pltpu.sync_copy(zeros_hbm, rows_v.at[pl.ds(0, s_per_t)])
        pltpu.sync_copy(rows_v.at[pl.ds(0, s_per_t)],
                        acc_sh.at[pl.ds(s * s_per_t, s_per_t)])
        plsc.subcore_barrier()
        pltpu.sync_copy(idx_hbm.at[wid], idx_v)
        pltpu.sync_copy(data_hbm.at[pl.ds(wid * b_per_w, b_per_w)], rows_v)
        for j in range(n_ch):
            pltpu.sync_copy(rows_v.at[pl.ds(j * CH, CH)],
                            acc_sh.at[idx_v.at[j]], add=True)
        plsc.subcore_barrier()
        pltpu.sync_copy(acc_sh.at[pl.ds(s * s_per_t, s_per_t)],
                        rows_v.at[pl.ds(0, s_per_t)])
        pltpu.sync_copy(rows_v.at[pl.ds(0, s_per_t)],
                        out_hbm.at[c].at[pl.ds(s * s_per_t, s_per_t)])

    return scatter


@functools.partial(
    pl.kernel,
    out_type=(jax.ShapeDtypeStruct((N_PAD, HID), _f32),
              jax.ShapeDtypeStruct((N_PAD, HID), _f32)),
    mesh=_mesh(),
    compiler_params=_SC_PARAMS,
    scratch_types=[
        pltpu.VMEM((N_SUB, 2 * HID), _f32),
        pltpu.VMEM((N_SUB, 2 * HID), _f32),
        pltpu.VMEM((N_SUB, 2 * HID), _f32),
        pltpu.VMEM((N_SUB, 2 * HID), _f32),
        pltpu.VMEM((N_SUB, HID), _f32),
        pltpu.VMEM((N_SUB, HID), _f32),
        pltpu.VMEM((N_SUB, HID), _f32),
        pltpu.SemaphoreType.DMA,
    ],
)
def _sc_node1(pa, pb, pre, x1_out, deg_out, q0, q1, q2, q3, pre_v, x1_v,
              deg_v, sem):
    wid = lax.axis_index("c") * NS + lax.axis_index("s")
    for sub in range(NSUB):
        base = wid * N_PER_T + sub * N_SUB
        pltpu.sync_copy(pa.at[0, pl.ds(base, N_SUB)], q0)
        pltpu.sync_copy(pa.at[1, pl.ds(base, N_SUB)], q1)
        pltpu.sync_copy(pb.at[0, pl.ds(base, N_SUB)], q2)
        pltpu.sync_copy(pb.at[1, pl.ds(base, N_SUB)], q3)
        pltpu.sync_copy(pre.at[pl.ds(base, N_SUB)], pre_v)

        def body(i4, _):
            for k in range(4):
                i = 4 * i4 + k
                agg = (q0[i, pl.ds(0, HID)] + q1[i, pl.ds(0, HID)]
                       + q2[i, pl.ds(0, HID)] + q3[i, pl.ds(0, HID)])
                deg = (q0[i, pl.ds(HID, HID)] + q1[i, pl.ds(HID, HID)]
                       + q2[i, pl.ds(HID, HID)] + q3[i, pl.ds(HID, HID)])
                x1_v[i] = jnp.maximum(
                    pre_v[i] + agg / jnp.maximum(deg, 1.0), 0.0)
                deg_v[i] = deg
            return 0

        lax.fori_loop(0, N_SUB // 4, body, 0)
        pltpu.sync_copy(x1_v, x1_out.at[pl.ds(base, N_SUB)])
        pltpu.sync_copy(deg_v, deg_out.at[pl.ds(base, N_SUB)])


@functools.partial(
    pl.kernel,
    out_type=jax.ShapeDtypeStruct((NC, N_GRAPHS, 48), _f32),
    mesh=_mesh(),
    compiler_params=_SC_PARAMS,
    scratch_types=[
        pltpu.VMEM((N_SUB, 2 * HID), _f32),
        pltpu.VMEM((N_SUB, 2 * HID), _f32),
        pltpu.VMEM((N_SUB, 2 * HID), _f32),
        pltpu.VMEM((N_SUB, 2 * HID), _f32),
        pltpu.VMEM((N_SUB, 2 * HID), _f32),
        pltpu.VMEM((N_SUB, HID), _f32),
        pltpu.VMEM((N_PER_T, 48), _f32),
        pltpu.VMEM((N_PER_T // CH, CH), jnp.int32),
        pltpu.VMEM_SHARED((N_GRAPHS, 48), _f32),
        pltpu.SemaphoreType.DMA,
    ],
)
def _sc_node2_pool(pa, pb, pre, degi, bidx, zeros_hbm, out, q0, q1, q2, q3,
                   pre_v, deg_v, haug, idx_v, acc_sh, sem):
    c = lax.axis_index("c")
    s = lax.axis_index("s")
    wid = c * NS + s
    spt = N_GRAPHS // NS
    pltpu.sync_copy(zeros_hbm, haug.at[pl.ds(0, spt)])
    pltpu.sync_copy(haug.at[pl.ds(0, spt)], acc_sh.at[pl.ds(s * spt, spt)])
    plsc.subcore_barrier()
    pltpu.sync_copy(bidx.at[wid], idx_v)
    for sub in range(NSUB):
        base = wid * N_PER_T + sub * N_SUB
        pltpu.sync_copy(pa.at[0, pl.ds(base, N_SUB)], q0)
        pltpu.sync_copy(pa.at[1, pl.ds(base, N_SUB)], q1)
        pltpu.sync_copy(pb.at[0, pl.ds(base, N_SUB)], q2)
        pltpu.sync_copy(pb.at[1, pl.ds(base, N_SUB)], q3)
        pltpu.sync_copy(pre.at[pl.ds(base, N_SUB)], pre_v)
        pltpu.sync_copy(degi.at[pl.ds(base, N_SUB)], deg_v)

        def body(i4, _):
            for k in range(4):
                i = 4 * i4 + k
                t0 = (q0[i, pl.ds(0, HID)] + q1[i, pl.ds(0, HID)]
                      + q2[i, pl.ds(0, HID)] + q3[i, pl.ds(0, HID)])
                t1 = (q0[i, pl.ds(HID, HID)] + q1[i, pl.ds(HID, HID)]
                      + q2[i, pl.ds(HID, HID)] + q3[i, pl.ds(HID, HID)])
                degc = jnp.maximum(deg_v[i], 1.0)
                h2a = jnp.maximum(pre_v[i, pl.ds(0, HID)] + t0 / degc, 0.0)
                h2b = jnp.maximum(pre_v[i, pl.ds(HID, HID)] + t1 / degc, 0.0)
                one = jnp.where(base + i < N_NODES, 1.0, 0.0)
                row = sub * N_SUB + i
                haug[row, pl.ds(0, HID)] = h2a * one
                haug[row, pl.ds(HID, HID)] = h2b * one
                haug[row, pl.ds(2 * HID, HID)] = jnp.full((HID,), 1.0, _f32) * one
            return 0

        lax.fori_loop(0, N_SUB // 4, body, 0)
    for j in range(N_PER_T // CH):
        pltpu.sync_copy(haug.at[pl.ds(j * CH, CH)],
                        acc_sh.at[idx_v.at[j]], add=True)
    plsc.subcore_barrier()
    pltpu.sync_copy(acc_sh.at[pl.ds(s * spt, spt)], haug.at[pl.ds(0, spt)])
    pltpu.sync_copy(haug.at[pl.ds(0, spt)], out.at[c].at[pl.ds(s * spt, spt)])


def _dot(a, b):
    return jnp.dot(a, b, preferred_element_type=_f32)


def _edge_body(ea_ref, xg_ref, wa_ref, ba_ref, wb_ref, bb_ref, rep_ref,
               sel_ref, out_ref, *, aug, row_base):
    pid = pl.program_id(0)
    z = jnp.maximum(_dot(ea_ref[...], wa_ref[...]) + ba_ref[...], 0.0)
    h = _dot(z.astype(jnp.bfloat16), wb_ref[...]) + bb_ref[...]
    xr = _dot(xg_ref[...].astype(jnp.bfloat16), rep_ref[...])
    m = _dot(xr * h, sel_ref[...])
    if aug:
        m = jnp.concatenate([m, jnp.ones((EB, HID), _f32)], axis=1)
    row = lax.broadcasted_iota(jnp.int32, m.shape, 0) + (row_base + pid * EB)
    out_ref[...] = jnp.where(row < N_EDGES, m, 0.0)


def _edge_call(ea, xg, wa, ba, wb, bb, rep, sel, out_w, aug, row_base):
    full_w = out_w + HID if aug else out_w
    body = functools.partial(_edge_body, aug=aug, row_base=row_base)
    blk_off = row_base // EB
    return pl.pallas_call(
        body,
        grid=(E_HALF // EB,),
        in_specs=[
            pl.BlockSpec((EB, D_EDGE), lambda i: (i + blk_off, 0)),
            pl.BlockSpec((EB, xg.shape[1]), lambda i: (i, 0)),
            pl.BlockSpec(wa.shape, lambda i: (0, 0)),
            pl.BlockSpec(ba.shape, lambda i: (0, 0)),
            pl.BlockSpec(wb.shape, lambda i: (0, 0)),
            pl.BlockSpec(bb.shape, lambda i: (0, 0)),
            pl.BlockSpec(rep.shape, lambda i: (0, 0)),
            pl.BlockSpec(sel.shape, lambda i: (0, 0)),
        ],
        out_specs=pl.BlockSpec((EB, full_w), lambda i: (i, 0)),
        out_shape=jax.ShapeDtypeStruct((E_HALF, full_w), _f32),
    )(ea, xg, wa, ba, wb, bb, rep, sel)


def _pre_body(x_ref, w_ref, b_ref, o_ref):
    o_ref[...] = _dot(x_ref[...], w_ref[...]) + b_ref[...]


def _pre_call(x, w, b):
    return pl.pallas_call(
        _pre_body,
        out_shape=jax.ShapeDtypeStruct((N_PAD, w.shape[1]), _f32),
    )(x, w, b)


def _head_body(p_ref, w1_ref, b1_ref, w2_ref, b2_ref, w3_ref, b3_ref,
               w4_ref, b4_ref, out_ref):
    t = p_ref[0] + p_ref[1]
    sums = t[:, :2 * HID]
    cnt = t[:, 2 * HID:2 * HID + 1]
    g = sums / jnp.maximum(cnt, 1.0)
    g = jnp.maximum(_dot(g, w1_ref[...]) + b1_ref[...], 0.0)
    g = jnp.maximum(_dot(g, w2_ref[...]) + b2_ref[...], 0.0)
    g = jnp.maximum(_dot(g, w3_ref[...]) + b3_ref[...], 0.0)
    out_ref[...] = _dot(g, w4_ref[...]) + b4_ref[...]


def _head_call(p, w1, b1, w2, b2, w3, b3, w4, b4):
    return pl.pallas_call(
        _head_body,
        out_shape=jax.ShapeDtypeStruct((N_GRAPHS, N_CLASSES), _f32),
    )(p, w1, b1, w2, b2, w3, b3, w4, b4)


def _rep_sel(in_ch, out_ch):
    k = in_ch * out_ch
    rep = np.zeros((in_ch, k), np.float32)
    rep[np.arange(k) // out_ch, np.arange(k)] = 1.0
    sel = np.zeros((k, out_ch), np.float32)
    sel[np.arange(k), np.arange(k) % out_ch] = 1.0
    return rep, sel


_REP1, _SEL1 = _rep_sel(D_NODE, HID)
_REP2, _SEL2 = _rep_sel(HID, 2 * HID)


def kernel(x, edge_index, edge_attr, batch, W1a, b1a, W1b, b1b, W2a, b2a,
           W2b, b2b, root1, bias1, root2, bias2, fcW1, fcb1, fcW2, fcb2,
           fcW3, fcb3, fcW4, fcb4):
    src = edge_index[0]
    dst = edge_index[1]
    ep = E_PAD - N_EDGES
    src_p = jnp.concatenate([src, jnp.zeros((ep,), jnp.int32)])
    dst_p = jnp.concatenate([dst, jnp.zeros((ep,), jnp.int32)])
    src3d = [src_p[h * E_HALF:(h + 1) * E_HALF].reshape(NW, -1, CH) for h in (0, 1)]
    dst3d = [dst_p[h * E_HALF:(h + 1) * E_HALF].reshape(NW, -1, CH) for h in (0, 1)]
    ea_p = jnp.concatenate([edge_attr, jnp.zeros((ep, D_EDGE), _f32)])
    npd = N_PAD - N_NODES
    x_p = jnp.concatenate([x, jnp.zeros((npd, D_NODE), _f32)])
    batch3d = jnp.concatenate([batch, jnp.zeros((npd,), jnp.int32)]).reshape(NW, -1, CH)

    z_node = jnp.zeros((N_PAD // NS, 2 * HID), _f32)
    z_pool = jnp.zeros((N_GRAPHS // NS, 48), _f32)

    b1a_, b1b_, b2a_, b2b_ = (b.reshape(1, -1) for b in (b1a, b1b, b2a, b2b))
    fcb1_, fcb2_, fcb3_, fcb4_ = (b.reshape(1, -1) for b in (fcb1, fcb2, fcb3, fcb4))

    gather1 = _make_sc_gather(N_PAD, D_NODE, E_HALF)
    gather2 = _make_sc_gather(N_PAD, HID, E_HALF, dtype=_f32)
    scat_node = _make_sc_scatter(E_HALF, 2 * HID, N_PAD)

    bf16 = jnp.bfloat16
    x_pb = x_p.astype(bf16)
    w1b_b = W1b.astype(bf16)
    w2b_b = W2b.astype(bf16)
    b1b_b, b2b_b = b1b_.astype(bf16), b2b_.astype(bf16)
    rep1_b = jnp.asarray(_REP1, bf16)
    rep2_b = jnp.asarray(_REP2, bf16)
    sel1_b = jnp.asarray(_SEL1, bf16)
    sel2_b = jnp.asarray(_SEL2, bf16)

    pre1 = _pre_call(x_p, root1, bias1.reshape(1, -1))
    xg1 = [gather1(x_pb, s3) for s3 in src3d]
    msg1 = [_edge_call(ea_p, xg1[h], W1a, b1a_, w1b_b, b1b_b, rep1_b, _SEL1,
                       out_w=HID, aug=True, row_base=h * E_HALF) for h in (0, 1)]
    p1 = [scat_node(msg1[h], dst3d[h], z_node) for h in (0, 1)]
    x1, degv = _sc_node1(p1[0], p1[1], pre1)

    pre2 = _pre_call(x1, root2, bias2.reshape(1, -1))
    xg2 = [gather2(x1, s3) for s3 in src3d]
    msg2 = [_edge_call(ea_p, xg2[h], W2a, b2a_, w2b_b, b2b_b, rep2_b, _SEL2,
                       out_w=2 * HID, aug=False, row_base=h * E_HALF) for h in (0, 1)]
    p2 = [scat_node(msg2[h], dst3d[h], z_node) for h in (0, 1)]

    gp = _sc_node2_pool(p2[0], p2[1], pre2, degv, batch3d, z_pool)
    return _head_call(gp, fcW1, fcb1_, fcW2, fcb2_, fcW3, fcb3_, fcW4, fcb4_)


# speedup vs baseline: 2.7139x; 1.0157x over previous
import functools

import jax
import jax.numpy as jnp
import numpy as np
from jax import lax
from jax.experimental import pallas as pl
from jax.experimental.pallas import tpu as pltpu
from jax.experimental.pallas import tpu_sc as plsc

N_NODES = 20000
N_EDGES = 80000
N_GRAPHS = 512
D_NODE = 32
D_EDGE = 16
HID = 16
N_CLASSES = 10

NC = 2
NS = 16
NW = NC * NS
CH = 128

E_PAD = 81920
E_HALF = E_PAD // 2
N_PAD = 20480
N_PER_T = N_PAD // NW
NSUB = 2
N_SUB = N_PER_T // NSUB

EB = 4096

_f32 = jnp.float32


def _mesh():
    return plsc.VectorSubcoreMesh(core_axis_name="c", subcore_axis_name="s")


_SC_PARAMS = pltpu.CompilerParams(use_tc_tiling_on_sc=False)


def _make_sc_gather(n_rows, d, n_idx, dtype=jnp.bfloat16):
    b_per_w = n_idx // NW
    n_ch = b_per_w // CH

    @functools.partial(
        pl.kernel,
        out_type=jax.ShapeDtypeStruct((n_idx, d), dtype),
        mesh=_mesh(),
        compiler_params=_SC_PARAMS,
        scratch_types=[
            pltpu.VMEM((n_ch, CH), jnp.int32),
            pltpu.VMEM((b_per_w, d), dtype),
            pltpu.SemaphoreType.DMA,
        ],
    )
    def gather(table_hbm, idx_hbm, out_hbm, idx_v, rows_v, sem):
        wid = lax.axis_index("c") * NS + lax.axis_index("s")
        pltpu.sync_copy(idx_hbm.at[wid], idx_v)
        cps = [
            pltpu.async_copy(table_hbm.at[idx_v.at[j]],
                             rows_v.at[pl.ds(j * CH, CH)], sem)
            for j in range(n_ch)
        ]
        for cp in cps:
            cp.wait()
        pltpu.sync_copy(rows_v, out_hbm.at[pl.ds(wid * b_per_w, b_per_w)])

    return gather


def _make_sc_scatter(n_rows, w, n_seg):
    b_per_w = n_rows // NW
    n_ch = b_per_w // CH
    s_per_t = n_seg // NS

    @functools.partial(
        pl.kernel,
        out_type=jax.ShapeDtypeStruct((NC, n_seg, w), _f32),
        mesh=_mesh(),
        compiler_params=_SC_PARAMS,
        scratch_types=[
            pltpu.VMEM((n_ch, CH), jnp.int32),
            pltpu.VMEM((b_per_w, w), _f32),
            pltpu.VMEM_SHARED((n_seg, w), _f32),
            pltpu.SemaphoreType.DMA,
        ],
    )
    def scatter(data_hbm, idx_hbm, zeros_hbm, out_hbm, idx_v, rows_v, acc_sh, sem):
        c = lax.axis_index("c")
        s = lax.axis_index("s")
        wid = c * NS + s
        pltpu.sync_copy(zeros_hbm, rows_v.at[pl.ds(0, s_per_t)])
        pltpu.sync_copy(rows_v.at[pl.ds(0, s_per_t)],
                        acc_sh.at[pl.ds(s * s_per_t, s_per_t)])
        plsc.subcore_barrier()
        pltpu.sync_copy(idx_hbm.at[wid], idx_v)
        pltpu.sync_copy(data_hbm.at[pl.ds(wid * b_per_w, b_per_w)], rows_v)
        for j in range(n_ch):
            pltpu.sync_copy(rows_v.at[pl.ds(j * CH, CH)],
                            acc_sh.at[idx_v.at[j]], add=True)
        plsc.subcore_barrier()
        pltpu.sync_copy(acc_sh.at[pl.ds(s * s_per_t, s_per_t)],
                        rows_v.at[pl.ds(0, s_per_t)])
        pltpu.sync_copy(rows_v.at[pl.ds(0, s_per_t)],
                        out_hbm.at[c].at[pl.ds(s * s_per_t, s_per_t)])

    return scatter


@functools.partial(
    pl.kernel,
    out_type=(jax.ShapeDtypeStruct((N_PAD, HID), _f32),
              jax.ShapeDtypeStruct((N_PAD, HID), _f32)),
    mesh=_mesh(),
    compiler_params=_SC_PARAMS,
    scratch_types=[
        pltpu.VMEM((N_SUB, 2 * HID), _f32),
        pltpu.VMEM((N_SUB, 2 * HID), _f32),
        pltpu.VMEM((N_SUB, 2 * HID), _f32),
        pltpu.VMEM((N_SUB, 2 * HID), _f32),
        pltpu.VMEM((N_SUB, HID), _f32),
        pltpu.VMEM((N_SUB, HID), _f32),
        pltpu.VMEM((N_SUB, HID), _f32),
        pltpu.SemaphoreType.DMA,
    ],
)
def _sc_node1(pa, pb, pre, x1_out, deg_out, q0, q1, q2, q3, pre_v, x1_v,
              deg_v, sem):
    wid = lax.axis_index("c") * NS + lax.axis_index("s")
    for sub in range(NSUB):
        base = wid * N_PER_T + sub * N_SUB
        pltpu.sync_copy(pa.at[0, pl.ds(base, N_SUB)], q0)
        pltpu.sync_copy(pa.at[1, pl.ds(base, N_SUB)], q1)
        pltpu.sync_copy(pb.at[0, pl.ds(base, N_SUB)], q2)
        pltpu.sync_copy(pb.at[1, pl.ds(base, N_SUB)], q3)
        pltpu.sync_copy(pre.at[pl.ds(base, N_SUB)], pre_v)

        def body(i4, _):
            for k in range(4):
                i = 4 * i4 + k
                agg = (q0[i, pl.ds(0, HID)] + q1[i, pl.ds(0, HID)]
                       + q2[i, pl.ds(0, HID)] + q3[i, pl.ds(0, HID)])
                deg = (q0[i, pl.ds(HID, HID)] + q1[i, pl.ds(HID, HID)]
                       + q2[i, pl.ds(HID, HID)] + q3[i, pl.ds(HID, HID)])
                x1_v[i] = jnp.maximum(
                    pre_v[i] + agg / jnp.maximum(deg, 1.0), 0.0)
                deg_v[i] = deg
            return 0

        lax.fori_loop(0, N_SUB // 4, body, 0)
        pltpu.sync_copy(x1_v, x1_out.at[pl.ds(base, N_SUB)])
        pltpu.sync_copy(deg_v, deg_out.at[pl.ds(base, N_SUB)])


@functools.partial(
    pl.kernel,
    out_type=jax.ShapeDtypeStruct((NC, N_GRAPHS, 48), _f32),
    mesh=_mesh(),
    compiler_params=_SC_PARAMS,
    scratch_types=[
        pltpu.VMEM((N_SUB, 2 * HID), _f32),
        pltpu.VMEM((N_SUB, 2 * HID), _f32),
        pltpu.VMEM((N_SUB, 2 * HID), _f32),
        pltpu.VMEM((N_SUB, 2 * HID), _f32),
        pltpu.VMEM((N_SUB, 2 * HID), _f32),
        pltpu.VMEM((N_SUB, HID), _f32),
        pltpu.VMEM((N_PER_T, 48), _f32),
        pltpu.VMEM((N_PER_T // CH, CH), jnp.int32),
        pltpu.VMEM_SHARED((N_GRAPHS, 48), _f32),
        pltpu.SemaphoreType.DMA,
    ],
)
def _sc_node2_pool(pa, pb, pre, degi, bidx, zeros_hbm, out, q0, q1, q2, q3,
                   pre_v, deg_v, haug, idx_v, acc_sh, sem):
    c = lax.axis_index("c")
    s = lax.axis_index("s")
    wid = c * NS + s
    spt = N_GRAPHS // NS
    pltpu.sync_copy(zeros_hbm, haug.at[pl.ds(0, spt)])
    pltpu.sync_copy(haug.at[pl.ds(0, spt)], acc_sh.at[pl.ds(s * spt, spt)])
    plsc.subcore_barrier()
    pltpu.sync_copy(bidx.at[wid], idx_v)
    for sub in range(NSUB):
        base = wid * N_PER_T + sub * N_SUB
        pltpu.sync_copy(pa.at[0, pl.ds(base, N_SUB)], q0)
        pltpu.sync_copy(pa.at[1, pl.ds(base, N_SUB)], q1)
        pltpu.sync_copy(pb.at[0, pl.ds(base, N_SUB)], q2)
        pltpu.sync_copy(pb.at[1, pl.ds(base, N_SUB)], q3)
        pltpu.sync_copy(pre.at[pl.ds(base, N_SUB)], pre_v)
        pltpu.sync_copy(degi.at[pl.ds(base, N_SUB)], deg_v)

        def body(i4, _):
            for k in range(4):
                i = 4 * i4 + k
                t0 = (q0[i, pl.ds(0, HID)] + q1[i, pl.ds(0, HID)]
                      + q2[i, pl.ds(0, HID)] + q3[i, pl.ds(0, HID)])
                t1 = (q0[i, pl.ds(HID, HID)] + q1[i, pl.ds(HID, HID)]
                      + q2[i, pl.ds(HID, HID)] + q3[i, pl.ds(HID, HID)])
                degc = jnp.maximum(deg_v[i], 1.0)
                h2a = jnp.maximum(pre_v[i, pl.ds(0, HID)] + t0 / degc, 0.0)
                h2b = jnp.maximum(pre_v[i, pl.ds(HID, HID)] + t1 / degc, 0.0)
                one = jnp.where(base + i < N_NODES, 1.0, 0.0)
                row = sub * N_SUB + i
                haug[row, pl.ds(0, HID)] = h2a * one
                haug[row, pl.ds(HID, HID)] = h2b * one
                haug[row, pl.ds(2 * HID, HID)] = jnp.full((HID,), 1.0, _f32) * one
            return 0

        lax.fori_loop(0, N_SUB // 4, body, 0)
    for j in range(N_PER_T // CH):
        pltpu.sync_copy(haug.at[pl.ds(j * CH, CH)],
                        acc_sh.at[idx_v.at[j]], add=True)
    plsc.subcore_barrier()
    pltpu.sync_copy(acc_sh.at[pl.ds(s * spt, spt)], haug.at[pl.ds(0, spt)])
    pltpu.sync_copy(haug.at[pl.ds(0, spt)], out.at[c].at[pl.ds(s * spt, spt)])


def _dot(a, b):
    return jnp.dot(a, b, preferred_element_type=_f32)


def _edge_body(ea_ref, xg_ref, wa_ref, ba_ref, wb_ref, bb_ref, rep_ref,
               sel_ref, out_ref, *, aug, row_base):
    pid = pl.program_id(0)
    z = jnp.maximum(_dot(ea_ref[...], wa_ref[...]) + ba_ref[...], 0.0)
    h = _dot(z.astype(jnp.bfloat16), wb_ref[...]) + bb_ref[...]
    xr = _dot(xg_ref[...].astype(jnp.bfloat16), rep_ref[...])
    m = _dot(xr * h, sel_ref[...])
    if aug:
        m = jnp.concatenate([m, jnp.ones((EB, HID), _f32)], axis=1)
    row = lax.broadcasted_iota(jnp.int32, m.shape, 0) + (row_base + pid * EB)
    out_ref[...] = jnp.where(row < N_EDGES, m, 0.0)


def _edge_call(ea, xg, wa, ba, wb, bb, rep, sel, out_w, aug, row_base):
    full_w = out_w + HID if aug else out_w
    body = functools.partial(_edge_body, aug=aug, row_base=row_base)
    blk_off = row_base // EB
    return pl.pallas_call(
        body,
        grid=(E_HALF // EB,),
        in_specs=[
            pl.BlockSpec((EB, D_EDGE), lambda i: (i + blk_off, 0)),
            pl.BlockSpec((EB, xg.shape[1]), lambda i: (i, 0)),
            pl.BlockSpec(wa.shape, lambda i: (0, 0)),
            pl.BlockSpec(ba.shape, lambda i: (0, 0)),
            pl.BlockSpec(wb.shape, lambda i: (0, 0)),
            pl.BlockSpec(bb.shape, lambda i: (0, 0)),
            pl.BlockSpec(rep.shape, lambda i: (0, 0)),
            pl.BlockSpec(sel.shape, lambda i: (0, 0)),
        ],
        out_specs=pl.BlockSpec((EB, full_w), lambda i: (i, 0)),
        out_shape=jax.ShapeDtypeStruct((E_HALF, full_w), _f32),
    )(ea, xg, wa, ba, wb, bb, rep, sel)


def _pre_body(x_ref, w_ref, b_ref, o_ref):
    o_ref[...] = _dot(x_ref[...], w_ref[...]) + b_ref[...]


def _pre_call(x, w, b):
    return pl.pallas_call(
        _pre_body,
        out_shape=jax.ShapeDtypeStruct((N_PAD, w.shape[1]), _f32),
    )(x, w, b)


def _head_body(p_ref, w1_ref, b1_ref, w2_ref, b2_ref, w3_ref, b3_ref,
               w4_ref, b4_ref, out_ref):
    t = p_ref[0] + p_ref[1]
    sums = t[:, :2 * HID]
    cnt = t[:, 2 * HID:2 * HID + 1]
    g = sums / jnp.maximum(cnt, 1.0)
    g = jnp.maximum(_dot(g, w1_ref[...]) + b1_ref[...], 0.0)
    g = jnp.maximum(_dot(g, w2_ref[...]) + b2_ref[...], 0.0)
    g = jnp.maximum(_dot(g, w3_ref[...]) + b3_ref[...], 0.0)
    out_ref[...] = _dot(g, w4_ref[...]) + b4_ref[...]


def _head_call(p, w1, b1, w2, b2, w3, b3, w4, b4):
    return pl.pallas_call(
        _head_body,
        out_shape=jax.ShapeDtypeStruct((N_GRAPHS, N_CLASSES), _f32),
    )(p, w1, b1, w2, b2, w3, b3, w4, b4)


def _rep_sel(in_ch, out_ch):
    k = in_ch * out_ch
    rep = np.zeros((in_ch, k), np.float32)
    rep[np.arange(k) // out_ch, np.arange(k)] = 1.0
    sel = np.zeros((k, out_ch), np.float32)
    sel[np.arange(k), np.arange(k) % out_ch] = 1.0
    return rep, sel


_REP1, _SEL1 = _rep_sel(D_NODE, HID)
_REP2, _SEL2 = _rep_sel(HID, 2 * HID)


def kernel(x, edge_index, edge_attr, batch, W1a, b1a, W1b, b1b, W2a, b2a,
           W2b, b2b, root1, bias1, root2, bias2, fcW1, fcb1, fcW2, fcb2,
           fcW3, fcb3, fcW4, fcb4):
    src = edge_index[0]
    dst = edge_index[1]
    ep = E_PAD - N_EDGES
    src_p = jnp.concatenate([src, jnp.zeros((ep,), jnp.int32)])
    dst_p = jnp.concatenate([dst, jnp.zeros((ep,), jnp.int32)])
    src3d = [src_p[h * E_HALF:(h + 1) * E_HALF].reshape(NW, -1, CH) for h in (0, 1)]
    dst3d = [dst_p[h * E_HALF:(h + 1) * E_HALF].reshape(NW, -1, CH) for h in (0, 1)]
    ea_p = jnp.concatenate([edge_attr, jnp.zeros((ep, D_EDGE), _f32)])
    npd = N_PAD - N_NODES
    x_p = jnp.concatenate([x, jnp.zeros((npd, D_NODE), _f32)])
    batch3d = jnp.concatenate([batch, jnp.zeros((npd,), jnp.int32)]).reshape(NW, -1, CH)

    z_node = jnp.zeros((N_PAD // NS, 2 * HID), _f32)
    z_pool = jnp.zeros((N_GRAPHS // NS, 48), _f32)

    b1a_, b1b_, b2a_, b2b_ = (b.reshape(1, -1) for b in (b1a, b1b, b2a, b2b))
    fcb1_, fcb2_, fcb3_, fcb4_ = (b.reshape(1, -1) for b in (fcb1, fcb2, fcb3, fcb4))

    gather1 = _make_sc_gather(N_PAD, D_NODE, E_HALF)
    gather2 = _make_sc_gather(N_PAD, HID, E_HALF, dtype=_f32)
    scat_node = _make_sc_scatter(E_HALF, 2 * HID, N_PAD)

    bf16 = jnp.bfloat16
    x_pb = x_p.astype(bf16)
    w1b_b = W1b.astype(bf16)
    w2b_b = W2b.astype(bf16)
    b1b_b, b2b_b = b1b_.astype(bf16), b2b_.astype(bf16)
    rep1_b = jnp.asarray(_REP1, bf16)
    rep2_b = jnp.asarray(_REP2, bf16)
    sel1_b = jnp.asarray(_SEL1, bf16)
    sel2_b = jnp.asarray(_SEL2, bf16)

    pre1 = _pre_call(x_p, root1, bias1.reshape(1, -1))
    xg1 = [gather1(x_pb, s3) for s3 in src3d]
    msg1 = [_edge_call(ea_p, xg1[h], W1a, b1a_, w1b_b, b1b_b, rep1_b, _SEL1,
                       out_w=HID, aug=True, row_base=h * E_HALF) for h in (0, 1)]
    p1 = [scat_node(msg1[h], dst3d[h], z_node) for h in (0, 1)]
    x1, degv = _sc_node1(p1[0], p1[1], pre1)

    pre2 = _pre_call(x1, root2, bias2.reshape(1, -1))
    xg2 = [gather2(x1, s3) for s3 in src3d]
    msg2 = [_edge_call(ea_p, xg2[h], W2a, b2a_, w2b_b, b2b_b, rep2_b, _SEL2,
                       out_w=2 * HID, aug=False, row_base=h * E_HALF) for h in (0, 1)]
    p2 = [scat_node(msg2[h], dst3d[h], z_node) for h in (0, 1)]

    gp = _sc_node2_pool(p2[0], p2[1], pre2, degv, batch3d, z_pool)
    return _head_call(gp, fcW1, fcb1_, fcW2, fcb2_, fcW3, fcb3_, fcW4, fcb4_)
